# baseline (speedup 1.0000x reference)
import functools

import jax
import jax.numpy as jnp
from jax import lax
from jax.experimental import pallas as pl
from jax.experimental.pallas import tpu as pltpu
from jax.experimental.pallas import tpu_sc as plsc

_S = 200
_D = 128
_ROWS_PER_WORKER = 8
_NUM_WORKERS = _S // _ROWS_PER_WORKER


def _sc_combine_body(tt_hbm, st_hbm, lt_hbm, ti_hbm, si_hbm, li_hbm, b_hbm,
                     c_hbm, ti_v, si_v, li_v, rt_v, rs_v, rl_v, out_v, b_v,
                     sem):
    num_cores = 2
    wid = lax.axis_index("s") * num_cores + lax.axis_index("c")

    @pl.when(wid < _NUM_WORKERS)
    def _():
        base = wid * _ROWS_PER_WORKER
        pltpu.sync_copy(ti_hbm.at[pl.ds(base, _ROWS_PER_WORKER)], ti_v)
        pltpu.sync_copy(si_hbm.at[pl.ds(base, _ROWS_PER_WORKER)], si_v)
        pltpu.sync_copy(li_hbm.at[pl.ds(base, _ROWS_PER_WORKER)], li_v)
        pltpu.sync_copy(b_hbm, b_v)
        pltpu.async_copy(tt_hbm.at[ti_v], rt_v, sem).wait()
        pltpu.async_copy(st_hbm.at[si_v], rs_v, sem).wait()
        pltpu.async_copy(lt_hbm.at[li_v], rl_v, sem).wait()
        for r in range(_ROWS_PER_WORKER):
            for k in range(_D // 16):
                sl = pl.ds(k * 16, 16)
                out_v[r, sl] = (rt_v[r, sl] + rs_v[r, sl] + rl_v[r, sl]
                                + b_v[sl])
        pltpu.sync_copy(out_v, c_hbm.at[pl.ds(base, _ROWS_PER_WORKER)])


def _make_sc_combine():
    return pl.kernel(
        _sc_combine_body,
        out_type=jax.ShapeDtypeStruct((_S, _D), jnp.float32),
        mesh=plsc.VectorSubcoreMesh(
            core_axis_name="c", subcore_axis_name="s",
            num_cores=2, num_subcores=16),
        scratch_types=[
            pltpu.VMEM((_ROWS_PER_WORKER,), jnp.int32),
            pltpu.VMEM((_ROWS_PER_WORKER,), jnp.int32),
            pltpu.VMEM((_ROWS_PER_WORKER,), jnp.int32),
            pltpu.VMEM((_ROWS_PER_WORKER, _D), jnp.float32),
            pltpu.VMEM((_ROWS_PER_WORKER, _D), jnp.float32),
            pltpu.VMEM((_ROWS_PER_WORKER, _D), jnp.float32),
            pltpu.VMEM((_ROWS_PER_WORKER, _D), jnp.float32),
            pltpu.VMEM((_D,), jnp.float32),
            pltpu.SemaphoreType.DMA,
        ],
    )


def _fuse_body(tv_ref, w_ref, c_ref, out_ref):
    BB, S = tv_ref.shape
    tv = tv_ref[...].reshape(BB, S, 1)
    w = w_ref[...][None]
    c = c_ref[...][None]
    out_ref[...] = tv * w + c


def kernel(token_values, W_val, b_val, type_table, side_table, slot_table,
           token_type_ids, token_side_ids, token_slot_ids):
    B, S = token_values.shape
    D = W_val.shape[0]

    w_row = W_val.reshape(1, D)

    combined = _make_sc_combine()(type_table, side_table, slot_table,
                                  token_type_ids, token_side_ids,
                                  token_slot_ids, b_val)

    BB = 128
    fuse = pl.pallas_call(
        _fuse_body,
        grid=(B // BB,),
        in_specs=[
            pl.BlockSpec((BB, S), lambda i: (i, 0)),
            pl.BlockSpec((1, D), lambda i: (0, 0)),
            pl.BlockSpec((S, D), lambda i: (0, 0)),
        ],
        out_specs=pl.BlockSpec((BB, S, D), lambda i: (i, 0, 0)),
        out_shape=jax.ShapeDtypeStruct((B, S, D), jnp.float32),
        compiler_params=pltpu.CompilerParams(
            dimension_semantics=("parallel",)),
    )(token_values, w_row, combined)

    return fuse


# speedup vs baseline: 1.0163x; 1.0163x over previous
import functools

import jax
import jax.numpy as jnp
from jax import lax
from jax.experimental import pallas as pl
from jax.experimental.pallas import tpu as pltpu
from jax.experimental.pallas import tpu_sc as plsc

_S = 200
_D = 128
_ROWS_PER_WORKER = 8
_NUM_WORKERS = _S // _ROWS_PER_WORKER


def _sc_combine_body(tt_hbm, st_hbm, lt_hbm, ti_hbm, si_hbm, li_hbm,
                     c_hbm, ti_v, si_v, li_v, rt_v, rs_v, rl_v, out_v,
                     sem):
    num_cores = 2
    wid = lax.axis_index("s") * num_cores + lax.axis_index("c")

    @pl.when(wid < _NUM_WORKERS)
    def _():
        base = wid * _ROWS_PER_WORKER
        rows = pl.ds(base, _ROWS_PER_WORKER)
        cp1 = pltpu.async_copy(ti_hbm.at[rows], ti_v, sem)
        cp2 = pltpu.async_copy(si_hbm.at[rows], si_v, sem)
        cp3 = pltpu.async_copy(li_hbm.at[rows], li_v, sem)
        cp1.wait(); cp2.wait(); cp3.wait()
        g1 = pltpu.async_copy(tt_hbm.at[ti_v], rt_v, sem)
        g2 = pltpu.async_copy(st_hbm.at[si_v], rs_v, sem)
        g3 = pltpu.async_copy(lt_hbm.at[li_v], rl_v, sem)
        g1.wait(); g2.wait(); g3.wait()
        for r in range(_ROWS_PER_WORKER):
            for k in range(_D // 16):
                sl = pl.ds(k * 16, 16)
                out_v[r, sl] = rt_v[r, sl] + rs_v[r, sl] + rl_v[r, sl]
        pltpu.sync_copy(out_v, c_hbm.at[rows])


def _make_sc_combine():
    return pl.kernel(
        _sc_combine_body,
        out_type=jax.ShapeDtypeStruct((_S, _D), jnp.float32),
        mesh=plsc.VectorSubcoreMesh(
            core_axis_name="c", subcore_axis_name="s",
            num_cores=2, num_subcores=16),
        scratch_types=[
            pltpu.VMEM((_ROWS_PER_WORKER,), jnp.int32),
            pltpu.VMEM((_ROWS_PER_WORKER,), jnp.int32),
            pltpu.VMEM((_ROWS_PER_WORKER,), jnp.int32),
            pltpu.VMEM((_ROWS_PER_WORKER, _D), jnp.float32),
            pltpu.VMEM((_ROWS_PER_WORKER, _D), jnp.float32),
            pltpu.VMEM((_ROWS_PER_WORKER, _D), jnp.float32),
            pltpu.VMEM((_ROWS_PER_WORKER, _D), jnp.float32),
            pltpu.SemaphoreType.DMA,
        ],
    )


def _fuse_body(tv_ref, w_ref, b_ref, c_ref, out_ref):
    BB, S = tv_ref.shape
    tv = tv_ref[...].reshape(BB, S, 1)
    w = w_ref[...][None]
    cb = (c_ref[...] + b_ref[...])[None]
    out_ref[...] = tv * w + cb


def kernel(token_values, W_val, b_val, type_table, side_table, slot_table,
           token_type_ids, token_side_ids, token_slot_ids):
    B, S = token_values.shape
    D = W_val.shape[0]

    w_row = W_val.reshape(1, D)

    combined = _make_sc_combine()(type_table, side_table, slot_table,
                                  token_type_ids, token_side_ids,
                                  token_slot_ids)
    b_row = b_val.reshape(1, D)

    BB = 128
    fuse = pl.pallas_call(
        _fuse_body,
        grid=(B // BB,),
        in_specs=[
            pl.BlockSpec((BB, S), lambda i: (i, 0)),
            pl.BlockSpec((1, D), lambda i: (0, 0)),
            pl.BlockSpec((1, D), lambda i: (0, 0)),
            pl.BlockSpec((S, D), lambda i: (0, 0)),
        ],
        out_specs=pl.BlockSpec((BB, S, D), lambda i: (i, 0, 0)),
        out_shape=jax.ShapeDtypeStruct((B, S, D), jnp.float32),
        compiler_params=pltpu.CompilerParams(
            dimension_semantics=("parallel",)),
    )(token_values, w_row, b_row, combined)

    return fuse
